# baseline (speedup 1.0000x reference)
import jax
import jax.numpy as jnp
from jax.experimental import pallas as pl
from jax.experimental.pallas import tpu as pltpu

MODEL_DIM = 768
VOCAB = 16000
F_SP = 4000
F_MIN = 2000
HIGH_T = 0.7
MED_T = 0.3
S = 2048

VT = 256
N_VT = (VOCAB + VT - 1) // VT
RT = 256
F_SP_T = 1024
F_MIN_T = 512
N_FT = 4


def _strategy_body(att_ref, x_ref, cnt_ref, strat_ref, xbf_ref):
    ta = jnp.sum(att_ref[...], axis=1, keepdims=True)
    mx = jnp.max(ta)
    norm = ta / (mx + 1e-8)
    strat = jnp.where(norm >= HIGH_T, 2, jnp.where(norm >= MED_T, 1, 0))
    strat = strat.astype(jnp.int32)
    strat_ref[...] = jnp.broadcast_to(strat, (S, 128))
    xbf_ref[...] = x_ref[...].astype(jnp.bfloat16)
    cnt_ref[0] = jnp.sum((strat == 0).astype(jnp.int32))
    cnt_ref[1] = jnp.sum((strat == 1).astype(jnp.int32))
    cnt_ref[2] = jnp.sum((strat == 2).astype(jnp.int32))


def _stage1_body(x_ref, wsp1_ref, bsp1_ref, wmin_ref, bmin_ref, h1_ref, mf_ref):
    xb = x_ref[...].astype(jnp.bfloat16)
    a1 = jnp.dot(xb, wsp1_ref[...].astype(jnp.bfloat16),
                 preferred_element_type=jnp.float32) + bsp1_ref[...]
    h1 = 0.5 * a1 * (1.0 + jax.lax.erf(a1 / jnp.sqrt(2.0).astype(jnp.float32)))
    h1_ref[...] = h1.astype(jnp.bfloat16)
    a0 = jnp.dot(xb, wmin_ref[...].astype(jnp.bfloat16),
                 preferred_element_type=jnp.float32) + bmin_ref[...]
    mf_ref[...] = a0.astype(jnp.bfloat16)


def _proj_body(cnt_ref, strat_ref, xbf_ref, h1_ref, mf_ref, wf_ref, ws2_ref,
               bs2_ref, we_ref, be_ref, out_ref, wf_s, ws2_s, we_s):
    n0 = cnt_ref[0]
    n1 = cnt_ref[1]
    n2 = cnt_ref[2]

    @pl.when(n2 > 0)
    def _():
        wf_s[...] = wf_ref[...].astype(jnp.bfloat16)

    @pl.when(n1 > 0)
    def _():
        ws2_s[...] = ws2_ref[...].astype(jnp.bfloat16)

    @pl.when(n0 > 0)
    def _():
        we_s[...] = we_ref[...].astype(jnp.bfloat16)

    for i in range(S // RT):
        rs = pl.ds(i * RT, RT)
        s = strat_ref[rs, :][:, :1]
        out_ref[rs, :] = jnp.zeros((RT, VT), jnp.float32)

        @pl.when(n0 > 0)
        def _():
            mn = jnp.dot(mf_ref[rs, :], we_s[...],
                         preferred_element_type=jnp.float32) + be_ref[...]
            out_ref[rs, :] += jnp.where(s == 0, mn, 0.0)

        @pl.when(n1 > 0)
        def _():
            sp = jnp.dot(h1_ref[rs, :], ws2_s[...],
                         preferred_element_type=jnp.float32) + bs2_ref[...]
            out_ref[rs, :] += jnp.where(s == 1, sp, 0.0)

        @pl.when(n2 > 0)
        def _():
            fl = jnp.dot(xbf_ref[rs, :], wf_s[...],
                         preferred_element_type=jnp.float32)
            out_ref[rs, :] += jnp.where(s == 2, fl, 0.0)


def kernel(hidden_states, attention_weights, W_full, W_sp1, b_sp1, W_sp2, b_sp2,
           W_min, b_min, W_exp, b_exp):
    x = hidden_states.reshape(S, MODEL_DIM)
    att = attention_weights.reshape(S, -1)

    cnts, strat, xbf = pl.pallas_call(
        _strategy_body,
        out_shape=[
            jax.ShapeDtypeStruct((4,), jnp.int32),
            jax.ShapeDtypeStruct((S, 128), jnp.int32),
            jax.ShapeDtypeStruct((S, MODEL_DIM), jnp.bfloat16),
        ],
        out_specs=[
            pl.BlockSpec(memory_space=pltpu.SMEM),
            pl.BlockSpec(memory_space=pltpu.VMEM),
            pl.BlockSpec(memory_space=pltpu.VMEM),
        ],
        in_specs=[
            pl.BlockSpec(memory_space=pltpu.VMEM),
            pl.BlockSpec(memory_space=pltpu.VMEM),
        ],
    )(att, x)

    h1, mf = pl.pallas_call(
        _stage1_body,
        grid=(N_FT,),
        in_specs=[
            pl.BlockSpec((S, MODEL_DIM), lambda j: (0, 0)),
            pl.BlockSpec((MODEL_DIM, F_SP_T), lambda j: (0, j)),
            pl.BlockSpec((1, F_SP_T), lambda j: (0, j)),
            pl.BlockSpec((MODEL_DIM, F_MIN_T), lambda j: (0, j)),
            pl.BlockSpec((1, F_MIN_T), lambda j: (0, j)),
        ],
        out_specs=[
            pl.BlockSpec((S, F_SP_T), lambda j: (0, j)),
            pl.BlockSpec((S, F_MIN_T), lambda j: (0, j)),
        ],
        out_shape=[
            jax.ShapeDtypeStruct((S, F_SP), jnp.bfloat16),
            jax.ShapeDtypeStruct((S, F_MIN), jnp.bfloat16),
        ],
    )(x, W_sp1, b_sp1.reshape(1, -1), W_min, b_min.reshape(1, -1))

    out = pl.pallas_call(
        _proj_body,
        grid_spec=pltpu.PrefetchScalarGridSpec(
            num_scalar_prefetch=1,
            grid=(N_VT,),
            in_specs=[
                pl.BlockSpec((S, 128), lambda j, c: (0, 0)),
                pl.BlockSpec((S, MODEL_DIM), lambda j, c: (0, 0)),
                pl.BlockSpec((S, F_SP), lambda j, c: (0, 0)),
                pl.BlockSpec((S, F_MIN), lambda j, c: (0, 0)),
                pl.BlockSpec((MODEL_DIM, VT),
                             lambda j, c: (0, jnp.where(c[2] > 0, j, 0))),
                pl.BlockSpec((F_SP, VT),
                             lambda j, c: (0, jnp.where(c[1] > 0, j, 0))),
                pl.BlockSpec((1, VT), lambda j, c: (0, j)),
                pl.BlockSpec((F_MIN, VT),
                             lambda j, c: (0, jnp.where(c[0] > 0, j, 0))),
                pl.BlockSpec((1, VT), lambda j, c: (0, j)),
            ],
            out_specs=pl.BlockSpec((S, VT), lambda j, c: (0, j)),
            scratch_shapes=[
                pltpu.VMEM((MODEL_DIM, VT), jnp.bfloat16),
                pltpu.VMEM((F_SP, VT), jnp.bfloat16),
                pltpu.VMEM((F_MIN, VT), jnp.bfloat16),
            ],
        ),
        out_shape=jax.ShapeDtypeStruct((S, VOCAB), jnp.float32),
        compiler_params=pltpu.CompilerParams(
            vmem_limit_bytes=64 * 1024 * 1024),
    )(cnts, strat, xbf, h1, mf, W_full, W_sp2, b_sp2.reshape(1, -1),
      W_exp, b_exp.reshape(1, -1))

    return out.reshape(1, S, VOCAB)


# speedup vs baseline: 1.4806x; 1.4806x over previous
import jax
import jax.numpy as jnp
from jax.experimental import pallas as pl
from jax.experimental.pallas import tpu as pltpu

MODEL_DIM = 768
VOCAB = 16000
F_SP = 4000
F_MIN = 2000
HIGH_T = 0.7
MED_T = 0.3
S = 2048

VT = 256
N_VT = (VOCAB + VT - 1) // VT
RT = 256
F_SP_T = 1024
F_MIN_T = 512
N_FT = 4


def _strategy_body(att_ref, x_ref, cnt_ref, strat_ref, xbf_ref):
    ta = jnp.sum(att_ref[...], axis=1, keepdims=True)
    mx = jnp.max(ta)
    norm = ta / (mx + 1e-8)
    strat = jnp.where(norm >= HIGH_T, 2, jnp.where(norm >= MED_T, 1, 0))
    strat = strat.astype(jnp.int32)
    strat_ref[...] = jnp.broadcast_to(strat, (S, 128))
    xbf_ref[...] = x_ref[...].astype(jnp.bfloat16)
    cnt_ref[0] = jnp.sum((strat == 0).astype(jnp.int32))
    cnt_ref[1] = jnp.sum((strat == 1).astype(jnp.int32))
    cnt_ref[2] = jnp.sum((strat == 2).astype(jnp.int32))


def _stage1_body(x_ref, wsp1_ref, bsp1_ref, wmin_ref, bmin_ref, h1_ref, mf_ref):
    xb = x_ref[...].astype(jnp.bfloat16)
    a1 = jnp.dot(xb, wsp1_ref[...].astype(jnp.bfloat16),
                 preferred_element_type=jnp.float32) + bsp1_ref[...]
    h1 = 0.5 * a1 * (1.0 + jax.lax.erf(a1 / jnp.sqrt(2.0).astype(jnp.float32)))
    h1_ref[...] = h1.astype(jnp.bfloat16)
    a0 = jnp.dot(xb, wmin_ref[...].astype(jnp.bfloat16),
                 preferred_element_type=jnp.float32) + bmin_ref[...]
    mf_ref[...] = a0.astype(jnp.bfloat16)


def _proj_body(cnt_ref, strat_ref, xbf_ref, h1_ref, mf_ref, wf_ref, ws2_ref,
               bs2_ref, we_ref, be_ref, out_ref, wf_s, ws2_s, we_s):
    n0 = cnt_ref[0]
    n1 = cnt_ref[1]
    n2 = cnt_ref[2]

    @pl.when(n2 > 0)
    def _():
        wf_s[...] = wf_ref[...].astype(jnp.bfloat16)

    @pl.when(n1 > 0)
    def _():
        ws2_s[...] = ws2_ref[...].astype(jnp.bfloat16)

    @pl.when(n0 > 0)
    def _():
        we_s[...] = we_ref[...].astype(jnp.bfloat16)

    out_ref[...] = jnp.zeros((S, VT), jnp.float32)

    @pl.when(n0 > 0)
    def _():
        for i in range(S // RT):
            rs = pl.ds(i * RT, RT)
            s = strat_ref[rs, :][:, :1]
            mn = jnp.dot(mf_ref[rs, :], we_s[...],
                         preferred_element_type=jnp.float32) + be_ref[...]
            out_ref[rs, :] += jnp.where(s == 0, mn, 0.0)

    @pl.when(n1 > 0)
    def _():
        for i in range(S // RT):
            rs = pl.ds(i * RT, RT)
            s = strat_ref[rs, :][:, :1]
            sp = jnp.dot(h1_ref[rs, :], ws2_s[...],
                         preferred_element_type=jnp.float32) + bs2_ref[...]
            out_ref[rs, :] += jnp.where(s == 1, sp, 0.0)

    @pl.when(n2 > 0)
    def _():
        for i in range(S // RT):
            rs = pl.ds(i * RT, RT)
            s = strat_ref[rs, :][:, :1]
            fl = jnp.dot(xbf_ref[rs, :], wf_s[...],
                         preferred_element_type=jnp.float32)
            out_ref[rs, :] += jnp.where(s == 2, fl, 0.0)


def kernel(hidden_states, attention_weights, W_full, W_sp1, b_sp1, W_sp2, b_sp2,
           W_min, b_min, W_exp, b_exp):
    x = hidden_states.reshape(S, MODEL_DIM)
    att = attention_weights.reshape(S, -1)

    cnts, strat, xbf = pl.pallas_call(
        _strategy_body,
        out_shape=[
            jax.ShapeDtypeStruct((4,), jnp.int32),
            jax.ShapeDtypeStruct((S, 128), jnp.int32),
            jax.ShapeDtypeStruct((S, MODEL_DIM), jnp.bfloat16),
        ],
        out_specs=[
            pl.BlockSpec(memory_space=pltpu.SMEM),
            pl.BlockSpec(memory_space=pltpu.VMEM),
            pl.BlockSpec(memory_space=pltpu.VMEM),
        ],
        in_specs=[
            pl.BlockSpec(memory_space=pltpu.VMEM),
            pl.BlockSpec(memory_space=pltpu.VMEM),
        ],
    )(att, x)

    h1, mf = pl.pallas_call(
        _stage1_body,
        grid=(N_FT,),
        in_specs=[
            pl.BlockSpec((S, MODEL_DIM), lambda j: (0, 0)),
            pl.BlockSpec((MODEL_DIM, F_SP_T), lambda j: (0, j)),
            pl.BlockSpec((1, F_SP_T), lambda j: (0, j)),
            pl.BlockSpec((MODEL_DIM, F_MIN_T), lambda j: (0, j)),
            pl.BlockSpec((1, F_MIN_T), lambda j: (0, j)),
        ],
        out_specs=[
            pl.BlockSpec((S, F_SP_T), lambda j: (0, j)),
            pl.BlockSpec((S, F_MIN_T), lambda j: (0, j)),
        ],
        out_shape=[
            jax.ShapeDtypeStruct((S, F_SP), jnp.bfloat16),
            jax.ShapeDtypeStruct((S, F_MIN), jnp.bfloat16),
        ],
    )(x, W_sp1, b_sp1.reshape(1, -1), W_min, b_min.reshape(1, -1))

    out = pl.pallas_call(
        _proj_body,
        grid_spec=pltpu.PrefetchScalarGridSpec(
            num_scalar_prefetch=1,
            grid=(N_VT,),
            in_specs=[
                pl.BlockSpec((S, 128), lambda j, c: (0, 0)),
                pl.BlockSpec((S, MODEL_DIM), lambda j, c: (0, 0)),
                pl.BlockSpec((S, F_SP), lambda j, c: (0, 0)),
                pl.BlockSpec((S, F_MIN), lambda j, c: (0, 0)),
                pl.BlockSpec((MODEL_DIM, VT),
                             lambda j, c: (0, jnp.where(c[2] > 0, j, 0))),
                pl.BlockSpec((F_SP, VT),
                             lambda j, c: (0, jnp.where(c[1] > 0, j, 0))),
                pl.BlockSpec((1, VT), lambda j, c: (0, j)),
                pl.BlockSpec((F_MIN, VT),
                             lambda j, c: (0, jnp.where(c[0] > 0, j, 0))),
                pl.BlockSpec((1, VT), lambda j, c: (0, j)),
            ],
            out_specs=pl.BlockSpec((S, VT), lambda j, c: (0, j)),
            scratch_shapes=[
                pltpu.VMEM((MODEL_DIM, VT), jnp.bfloat16),
                pltpu.VMEM((F_SP, VT), jnp.bfloat16),
                pltpu.VMEM((F_MIN, VT), jnp.bfloat16),
            ],
        ),
        out_shape=jax.ShapeDtypeStruct((S, VOCAB), jnp.float32),
        compiler_params=pltpu.CompilerParams(
            vmem_limit_bytes=64 * 1024 * 1024),
    )(cnts, strat, xbf, h1, mf, W_full, W_sp2, b_sp2.reshape(1, -1),
      W_exp, b_exp.reshape(1, -1))

    return out.reshape(1, S, VOCAB)
